# baseline (speedup 1.0000x reference)
import functools

import jax
import jax.numpy as jnp
from jax import lax
from jax.experimental import pallas as pl
from jax.experimental.pallas import tpu as pltpu
from jax.experimental.pallas import tpu_sc as plsc

_DIM = 128
_MARGIN = 6.0
_EPSILON = 2.0
_PI = 3.141592653589793

_NC = 2
_NS = 16
_NW = _NC * _NS


def _pipelined_gather(streams, gsems, wsems, n_ch, n_slots):
    gath = {}
    writes = {}

    def start_gather(ci):
        slot = ci % n_slots
        gath[ci] = [
            pltpu.async_copy(
                s["table"].at[s["idx"].at[pl.ds(ci * s["C"], s["C"])]],
                s["bufs"][slot], gsems[slot])
            for s in streams]

    for ci in range(min(n_slots - 1, n_ch)):
        start_gather(ci)
    for ci in range(n_ch):
        slot = ci % n_slots
        for c in gath.pop(ci):
            c.wait()
        writes[ci] = [
            pltpu.async_copy(
                s["bufs"][slot],
                s["dst"].at[pl.ds(s["base"] + ci * s["C"], s["C"])],
                wsems[slot])
            for s in streams]
        nxt = ci + n_slots - 1
        if nxt < n_ch:
            if nxt - n_slots >= 0:
                for c in writes.pop(nxt - n_slots):
                    c.wait()
            start_gather(nxt)
    for ci in sorted(writes):
        for c in writes[ci]:
            c.wait()


@functools.lru_cache(maxsize=None)
def _sc_gather_fn(B2, Brel, D2, F, DR):
    rows_w = B2 // _NW
    C = 64
    NSLOT = 2
    CR = C * Brel // B2
    rel_w = Brel // _NW
    n_ch = rows_w // C

    mesh = plsc.VectorSubcoreMesh(core_axis_name="c", subcore_axis_name="s")

    buf_types = []
    for _ in range(NSLOT):
        buf_types += [
            pltpu.VMEM((C, D2), jnp.float32),
            pltpu.VMEM((C, F), jnp.float32),
            pltpu.VMEM((C, F), jnp.float32),
            pltpu.VMEM((CR, DR), jnp.float32),
        ]

    @functools.partial(
        pl.kernel,
        mesh=mesh,
        out_type=[
            jax.ShapeDtypeStruct((B2, D2), jnp.float32),
            jax.ShapeDtypeStruct((B2, F), jnp.float32),
            jax.ShapeDtypeStruct((B2, F), jnp.float32),
            jax.ShapeDtypeStruct((Brel, DR), jnp.float32),
        ],
        scratch_types=(
            [pltpu.VMEM((rows_w,), jnp.int32),
             pltpu.VMEM((rel_w,), jnp.int32)]
            + buf_types
            + [pltpu.SemaphoreType.DMA] * (2 * NSLOT)
        ),
    )
    def gather_kernel(ent_hbm, vis_hbm, txt_hbm, rel_hbm, idx_hbm, ridx_hbm,
                      ge_hbm, gv_hbm, gt_hbm, rg_hbm,
                      idx_full, ridx_full, *rest):
        bufs = rest[:4 * NSLOT]
        gsems = rest[4 * NSLOT:4 * NSLOT + NSLOT]
        wsems = rest[4 * NSLOT + NSLOT:]
        wid = lax.axis_index("s") * _NC + lax.axis_index("c")
        base = pl.multiple_of(wid * rows_w, 8)
        rbase = pl.multiple_of(wid * rel_w, 8)
        pltpu.sync_copy(idx_hbm.at[pl.ds(base, rows_w)], idx_full)
        pltpu.sync_copy(ridx_hbm.at[pl.ds(rbase, rel_w)], ridx_full)
        streams = [
            dict(table=ent_hbm, dst=ge_hbm, idx=idx_full, base=base, C=C,
                 bufs=[bufs[4 * s + 0] for s in range(NSLOT)]),
            dict(table=vis_hbm, dst=gv_hbm, idx=idx_full, base=base, C=C,
                 bufs=[bufs[4 * s + 1] for s in range(NSLOT)]),
            dict(table=txt_hbm, dst=gt_hbm, idx=idx_full, base=base, C=C,
                 bufs=[bufs[4 * s + 2] for s in range(NSLOT)]),
            dict(table=rel_hbm, dst=rg_hbm, idx=ridx_full, base=rbase, C=CR,
                 bufs=[bufs[4 * s + 3] for s in range(NSLOT)]),
        ]
        _pipelined_gather(streams, list(gsems), list(wsems), n_ch, NSLOT)

    return gather_kernel


@functools.lru_cache(maxsize=None)
def _tc_score_fn(B, D2, F, DR):
    BLK = 512
    nblk = B // BLK
    inv3 = 1.0 / 3.0
    phase_scale = _PI * _DIM / (_MARGIN + _EPSILON)

    def body(he_ref, hv_ref, ht_ref, te_ref, tv_ref, tt_ref, r_ref,
             wv_ref, wt_ref, o_ref):
        wv = wv_ref[...].astype(jnp.bfloat16)
        wt = wt_ref[...].astype(jnp.bfloat16)

        def mix(e_ref, v_ref, t_ref):
            v = v_ref[...].astype(jnp.bfloat16)
            t = t_ref[...].astype(jnp.bfloat16)
            return (jnp.dot(v, wv, preferred_element_type=jnp.float32)
                    + jnp.dot(t, wt, preferred_element_type=jnp.float32)
                    + e_ref[...]) * inv3

        mh = mix(he_ref, hv_ref, ht_ref)
        mt = mix(te_ref, tv_ref, tt_ref)
        phase = r_ref[...] * phase_scale
        re_r = jnp.cos(phase)
        im_r = jnp.sin(phase)
        re_h, im_h = mh[:, :_DIM], mh[:, _DIM:]
        re_t, im_t = mt[:, :_DIM], mt[:, _DIM:]
        re_s = re_h * re_r - im_h * im_r - re_t
        im_s = re_h * im_r + im_h * re_r - im_t
        dist = jnp.sum(jnp.sqrt(re_s * re_s + im_s * im_s + 1e-12), axis=1)
        o_ref[...] = _MARGIN - dist

    return pl.pallas_call(
        body,
        grid=(nblk,),
        in_specs=[
            pl.BlockSpec((BLK, D2), lambda i: (i, 0)),
            pl.BlockSpec((BLK, F), lambda i: (i, 0)),
            pl.BlockSpec((BLK, F), lambda i: (i, 0)),
            pl.BlockSpec((BLK, D2), lambda i: (i + nblk, 0)),
            pl.BlockSpec((BLK, F), lambda i: (i + nblk, 0)),
            pl.BlockSpec((BLK, F), lambda i: (i + nblk, 0)),
            pl.BlockSpec((BLK, DR), lambda i: (i, 0)),
            pl.BlockSpec((F, D2), lambda i: (0, 0)),
            pl.BlockSpec((F, D2), lambda i: (0, 0)),
        ],
        out_specs=pl.BlockSpec((BLK,), lambda i: (i,)),
        out_shape=jax.ShapeDtypeStruct((B,), jnp.float32),
    )


_NSPLIT = 2


def kernel(ent_emb, rel_emb, vis_feats, txt_feats, W_vis, W_txt,
           head, relation, tail):
    B = head.shape[0]
    D2 = ent_emb.shape[1]
    F = vis_feats.shape[1]
    DR = rel_emb.shape[1]
    head = head.astype(jnp.int32)
    tail = tail.astype(jnp.int32)
    rel_idx = relation.astype(jnp.int32)
    Bp = B // _NSPLIT
    sc = _sc_gather_fn(2 * Bp, Bp, D2, F, DR)
    tc = _tc_score_fn(Bp, D2, F, DR)
    outs = []
    for p in range(_NSPLIT):
        sl = slice(p * Bp, (p + 1) * Bp)
        idx_p = jnp.concatenate([head[sl], tail[sl]])
        ge, gv, gt, rg = sc(ent_emb, vis_feats, txt_feats, rel_emb,
                            idx_p, rel_idx[sl])
        outs.append(tc(ge, gv, gt, ge, gv, gt, rg, W_vis, W_txt))
    return jnp.concatenate(outs)


# speedup vs baseline: 1.0289x; 1.0289x over previous
import functools

import jax
import jax.numpy as jnp
from jax import lax
from jax.experimental import pallas as pl
from jax.experimental.pallas import tpu as pltpu
from jax.experimental.pallas import tpu_sc as plsc

_DIM = 128
_MARGIN = 6.0
_EPSILON = 2.0
_PI = 3.141592653589793

_NC = 2
_NS = 16
_NW = _NC * _NS


def _pipelined_gather(streams, gsems, wsems, n_ch, n_slots):
    gath = {}
    writes = {}

    def start_gather(ci):
        slot = ci % n_slots
        gath[ci] = [
            pltpu.async_copy(
                s["table"].at[s["idx"].at[pl.ds(ci * s["C"], s["C"])]],
                s["bufs"][slot], gsems[slot])
            for s in streams]

    for ci in range(min(n_slots - 1, n_ch)):
        start_gather(ci)
    for ci in range(n_ch):
        slot = ci % n_slots
        for c in gath.pop(ci):
            c.wait()
        writes[ci] = [
            pltpu.async_copy(
                s["bufs"][slot],
                s["dst"].at[pl.ds(s["base"] + ci * s["C"], s["C"])],
                wsems[slot])
            for s in streams]
        nxt = ci + n_slots - 1
        if nxt < n_ch:
            if nxt - n_slots >= 0:
                for c in writes.pop(nxt - n_slots):
                    c.wait()
            start_gather(nxt)
    for ci in sorted(writes):
        for c in writes[ci]:
            c.wait()


@functools.lru_cache(maxsize=None)
def _sc_gather_fn(B2, Brel, D2, F, DR):
    rows_w = B2 // _NW
    C = 32
    NSLOT = 4
    CR = C * Brel // B2
    rel_w = Brel // _NW
    n_ch = rows_w // C

    mesh = plsc.VectorSubcoreMesh(core_axis_name="c", subcore_axis_name="s")

    buf_types = []
    for _ in range(NSLOT):
        buf_types += [
            pltpu.VMEM((C, D2), jnp.float32),
            pltpu.VMEM((C, F), jnp.float32),
            pltpu.VMEM((C, F), jnp.float32),
            pltpu.VMEM((CR, DR), jnp.float32),
        ]

    @functools.partial(
        pl.kernel,
        mesh=mesh,
        out_type=[
            jax.ShapeDtypeStruct((B2, D2), jnp.float32),
            jax.ShapeDtypeStruct((B2, F), jnp.float32),
            jax.ShapeDtypeStruct((B2, F), jnp.float32),
            jax.ShapeDtypeStruct((Brel, DR), jnp.float32),
        ],
        scratch_types=(
            [pltpu.VMEM((rows_w,), jnp.int32),
             pltpu.VMEM((rel_w,), jnp.int32)]
            + buf_types
            + [pltpu.SemaphoreType.DMA] * (2 * NSLOT)
        ),
    )
    def gather_kernel(ent_hbm, vis_hbm, txt_hbm, rel_hbm, idx_hbm, ridx_hbm,
                      ge_hbm, gv_hbm, gt_hbm, rg_hbm,
                      idx_full, ridx_full, *rest):
        bufs = rest[:4 * NSLOT]
        gsems = rest[4 * NSLOT:4 * NSLOT + NSLOT]
        wsems = rest[4 * NSLOT + NSLOT:]
        wid = lax.axis_index("s") * _NC + lax.axis_index("c")
        base = pl.multiple_of(wid * rows_w, 8)
        rbase = pl.multiple_of(wid * rel_w, 8)
        pltpu.sync_copy(idx_hbm.at[pl.ds(base, rows_w)], idx_full)
        pltpu.sync_copy(ridx_hbm.at[pl.ds(rbase, rel_w)], ridx_full)
        streams = [
            dict(table=ent_hbm, dst=ge_hbm, idx=idx_full, base=base, C=C,
                 bufs=[bufs[4 * s + 0] for s in range(NSLOT)]),
            dict(table=vis_hbm, dst=gv_hbm, idx=idx_full, base=base, C=C,
                 bufs=[bufs[4 * s + 1] for s in range(NSLOT)]),
            dict(table=txt_hbm, dst=gt_hbm, idx=idx_full, base=base, C=C,
                 bufs=[bufs[4 * s + 2] for s in range(NSLOT)]),
            dict(table=rel_hbm, dst=rg_hbm, idx=ridx_full, base=rbase, C=CR,
                 bufs=[bufs[4 * s + 3] for s in range(NSLOT)]),
        ]
        _pipelined_gather(streams, list(gsems), list(wsems), n_ch, NSLOT)

    return gather_kernel


@functools.lru_cache(maxsize=None)
def _tc_score_fn(B, D2, F, DR):
    BLK = 512
    nblk = B // BLK
    inv3 = 1.0 / 3.0
    phase_scale = _PI * _DIM / (_MARGIN + _EPSILON)

    def body(he_ref, hv_ref, ht_ref, te_ref, tv_ref, tt_ref, r_ref,
             wv_ref, wt_ref, o_ref):
        wv = wv_ref[...].astype(jnp.bfloat16)
        wt = wt_ref[...].astype(jnp.bfloat16)

        def mix(e_ref, v_ref, t_ref):
            v = v_ref[...].astype(jnp.bfloat16)
            t = t_ref[...].astype(jnp.bfloat16)
            return (jnp.dot(v, wv, preferred_element_type=jnp.float32)
                    + jnp.dot(t, wt, preferred_element_type=jnp.float32)
                    + e_ref[...]) * inv3

        mh = mix(he_ref, hv_ref, ht_ref)
        mt = mix(te_ref, tv_ref, tt_ref)
        phase = r_ref[...] * phase_scale
        re_r = jnp.cos(phase)
        im_r = jnp.sin(phase)
        re_h, im_h = mh[:, :_DIM], mh[:, _DIM:]
        re_t, im_t = mt[:, :_DIM], mt[:, _DIM:]
        re_s = re_h * re_r - im_h * im_r - re_t
        im_s = re_h * im_r + im_h * re_r - im_t
        dist = jnp.sum(jnp.sqrt(re_s * re_s + im_s * im_s + 1e-12), axis=1)
        o_ref[...] = _MARGIN - dist

    return pl.pallas_call(
        body,
        grid=(nblk,),
        in_specs=[
            pl.BlockSpec((BLK, D2), lambda i: (i, 0)),
            pl.BlockSpec((BLK, F), lambda i: (i, 0)),
            pl.BlockSpec((BLK, F), lambda i: (i, 0)),
            pl.BlockSpec((BLK, D2), lambda i: (i + nblk, 0)),
            pl.BlockSpec((BLK, F), lambda i: (i + nblk, 0)),
            pl.BlockSpec((BLK, F), lambda i: (i + nblk, 0)),
            pl.BlockSpec((BLK, DR), lambda i: (i, 0)),
            pl.BlockSpec((F, D2), lambda i: (0, 0)),
            pl.BlockSpec((F, D2), lambda i: (0, 0)),
        ],
        out_specs=pl.BlockSpec((BLK,), lambda i: (i,)),
        out_shape=jax.ShapeDtypeStruct((B,), jnp.float32),
    )


_NSPLIT = 2


def kernel(ent_emb, rel_emb, vis_feats, txt_feats, W_vis, W_txt,
           head, relation, tail):
    B = head.shape[0]
    D2 = ent_emb.shape[1]
    F = vis_feats.shape[1]
    DR = rel_emb.shape[1]
    head = head.astype(jnp.int32)
    tail = tail.astype(jnp.int32)
    rel_idx = relation.astype(jnp.int32)
    Bp = B // _NSPLIT
    sc = _sc_gather_fn(2 * Bp, Bp, D2, F, DR)
    tc = _tc_score_fn(Bp, D2, F, DR)
    outs = []
    for p in range(_NSPLIT):
        sl = slice(p * Bp, (p + 1) * Bp)
        idx_p = jnp.concatenate([head[sl], tail[sl]])
        ge, gv, gt, rg = sc(ent_emb, vis_feats, txt_feats, rel_emb,
                            idx_p, rel_idx[sl])
        outs.append(tc(ge, gv, gt, ge, gv, gt, rg, W_vis, W_txt))
    return jnp.concatenate(outs)
